# baseline (speedup 1.0000x reference)
import jax
import jax.numpy as jnp
from jax import lax
from jax.experimental import pallas as pl
from jax.experimental.pallas import tpu as pltpu
from jax.experimental.pallas import tpu_sc as plsc

_NUM_WORKERS = 32


def _worker_segments(n_rows: int, shift: int):
    split = n_rows - shift
    chunk = -(-n_rows // _NUM_WORKERS)
    segs = []
    for w in range(_NUM_WORKERS):
        o0 = min(w * chunk, n_rows)
        o1 = min(o0 + chunk, n_rows)
        s = []
        if o0 < split:
            a1 = min(o1, split)
            s.append((o0, o0 + shift, a1 - o0))
        if o1 > split:
            b0 = max(o0, split)
            s.append((b0, b0 - split, o1 - b0))
        segs.append(s)
    return segs


def kernel(x, table):
    seq_len = x.shape[1]
    n_rows, d = table.shape
    assert n_rows == 2 * seq_len - 1
    worker_segs = _worker_segments(n_rows, seq_len)

    def body(table_hbm, out_hbm):
        wid = lax.axis_index("s") * 2 + lax.axis_index("c")
        for w, segs in enumerate(worker_segs):
            @pl.when(wid == w)
            def _(segs=segs):
                for dst, src, n in segs:
                    pltpu.sync_copy(table_hbm.at[pl.ds(src, n)],
                                    out_hbm.at[pl.ds(dst, n)])

    f = pl.kernel(
        body,
        out_type=jax.ShapeDtypeStruct((n_rows, d), table.dtype),
        mesh=plsc.VectorSubcoreMesh(core_axis_name="c", subcore_axis_name="s"),
        compiler_params=pltpu.CompilerParams(use_tc_tiling_on_sc=False),
    )
    return f(table)


# speedup vs baseline: 11.0160x; 11.0160x over previous
import jax
import jax.numpy as jnp
from jax import lax
from jax.experimental import pallas as pl
from jax.experimental.pallas import tpu as pltpu
from jax.experimental.pallas import tpu_sc as plsc

_NW = 32
_CHUNK = 32
_NBUF = 3


def kernel(x, table):
    seq_len = x.shape[1]
    n_rows, d = table.shape
    assert n_rows == 2 * seq_len - 1
    per_w = seq_len // _NW
    assert per_w % _CHUNK == 0
    n_chunks = 2 * (per_w // _CHUNK)
    split = n_rows - seq_len

    def body(table_hbm, out_hbm, *scratch):
        bufs = scratch[:_NBUF]
        gsems = scratch[_NBUF:2 * _NBUF]
        ssems = scratch[2 * _NBUF:]
        wid = lax.axis_index("s") * 2 + lax.axis_index("c")

        a_dst = jnp.minimum(wid * per_w, split - per_w)
        b_dst = split + wid * per_w

        def offsets(i):
            off = (i % (n_chunks // 2)) * _CHUNK
            if i < n_chunks // 2:
                return a_dst + off, a_dst + seq_len + off
            return b_dst + off, (b_dst - split) + off

        def gather(i):
            _, src = offsets(i)
            b = i % _NBUF
            return pltpu.make_async_copy(
                table_hbm.at[pl.ds(src, _CHUNK)], bufs[b], gsems[b])

        def scatter(i):
            dst, _ = offsets(i)
            b = i % _NBUF
            return pltpu.make_async_copy(
                bufs[b], out_hbm.at[pl.ds(dst, _CHUNK)], ssems[b])

        for i in range(n_chunks):
            if i >= _NBUF:
                scatter(i - _NBUF).wait()
            gather(i).start()
            if i >= 1:
                gather(i - 1).wait()
                scatter(i - 1).start()
        gather(n_chunks - 1).wait()
        scatter(n_chunks - 1).start()
        for i in range(n_chunks - _NBUF + 1, n_chunks + 1):
            scatter(i - 1).wait()

    f = pl.kernel(
        body,
        out_type=jax.ShapeDtypeStruct((n_rows, d), table.dtype),
        mesh=plsc.VectorSubcoreMesh(core_axis_name="c", subcore_axis_name="s"),
        scratch_types=([pltpu.VMEM((_CHUNK, d), table.dtype)] * _NBUF
                       + [pltpu.SemaphoreType.DMA] * (2 * _NBUF)),
        compiler_params=pltpu.CompilerParams(use_tc_tiling_on_sc=False),
    )
    return f(table)


# speedup vs baseline: 32.7152x; 2.9698x over previous
import jax
import jax.numpy as jnp
from jax import lax
from jax.experimental import pallas as pl
from jax.experimental.pallas import tpu as pltpu
from jax.experimental.pallas import tpu_sc as plsc

_NW = 32
_CHUNK = 32
_NBUF = 3
_LANES = 16


def kernel(x, table):
    seq_len = x.shape[1]
    n_rows, d = table.shape
    assert n_rows == 2 * seq_len - 1 and seq_len % _NW == 0
    per_w = seq_len // _NW
    n_reg = per_w // _CHUNK
    split = seq_len - 1
    a_hi = (split // 8) * 8
    tail_dst0 = split - _LANES

    def body(table_hbm, out_hbm, *scratch):
        bufs = scratch[:_NBUF]
        idxs = scratch[_NBUF:2 * _NBUF]
        gsems = scratch[2 * _NBUF:3 * _NBUF]
        ssems = scratch[3 * _NBUF:4 * _NBUF]
        tbuf, tidx_g, tidx_s, tsem = scratch[4 * _NBUF:]
        wid = lax.axis_index("s") * 2 + lax.axis_index("c")
        iota = lax.iota(jnp.int32, _LANES)

        a_dst = jnp.minimum(wid * per_w, a_hi - per_w)
        b_src = wid * per_w

        def chunk_off(i):
            return (i % n_reg) * _CHUNK

        def gather(i):
            b = i % _NBUF
            if i < n_reg:
                src = pl.multiple_of(a_dst + seq_len + chunk_off(i), 8)
            else:
                src = pl.multiple_of(b_src + chunk_off(i), 8)
            return pltpu.make_async_copy(
                table_hbm.at[pl.ds(src, _CHUNK)], bufs[b], gsems[b])

        def fill_idx(i):
            b = i % _NBUF
            dst0 = b_src + chunk_off(i) + split
            for k in range(_CHUNK // _LANES):
                idxs[b][pl.ds(k * _LANES, _LANES)] = dst0 + k * _LANES + iota

        def scatter(i):
            b = i % _NBUF
            if i < n_reg:
                dst = pl.multiple_of(a_dst + chunk_off(i), 8)
                return pltpu.make_async_copy(
                    bufs[b], out_hbm.at[pl.ds(dst, _CHUNK)], ssems[b])
            return pltpu.make_async_copy(bufs[b], out_hbm.at[idxs[b]], ssems[b])

        n_chunks = 2 * n_reg
        for i in range(n_chunks):
            if i >= _NBUF:
                scatter(i - _NBUF).wait()
            gather(i).start()
            if i >= 1:
                gather(i - 1).wait()
                if i - 1 >= n_reg:
                    fill_idx(i - 1)
                scatter(i - 1).start()
        gather(n_chunks - 1).wait()
        fill_idx(n_chunks - 1)
        scatter(n_chunks - 1).start()
        for i in range(n_chunks - _NBUF, n_chunks):
            scatter(i).wait()

        @pl.when(wid == 0)
        def _():
            tidx_g[...] = tail_dst0 + seq_len + iota
            tidx_s[...] = tail_dst0 + iota
            tg = pltpu.make_async_copy(table_hbm.at[tidx_g], tbuf, tsem)
            tg.start()
            tg.wait()
            ts = pltpu.make_async_copy(tbuf, out_hbm.at[tidx_s], tsem)
            ts.start()
            ts.wait()

    f = pl.kernel(
        body,
        out_type=jax.ShapeDtypeStruct((n_rows, d), table.dtype),
        mesh=plsc.VectorSubcoreMesh(core_axis_name="c", subcore_axis_name="s"),
        scratch_types=([pltpu.VMEM((_CHUNK, d), table.dtype)] * _NBUF
                       + [pltpu.VMEM((_CHUNK,), jnp.int32)] * _NBUF
                       + [pltpu.SemaphoreType.DMA] * (2 * _NBUF)
                       + [pltpu.VMEM((_LANES, d), table.dtype),
                          pltpu.VMEM((_LANES,), jnp.int32),
                          pltpu.VMEM((_LANES,), jnp.int32),
                          pltpu.SemaphoreType.DMA]),
    )
    return f(table)
